# baseline (speedup 1.0000x reference)
import jax
import jax.numpy as jnp
from jax import lax
from jax.experimental import pallas as pl
from jax.experimental.pallas import tpu as pltpu
from jax.experimental.pallas import tpu_sc as plsc

_NC = 2
_NS = 16
_NW = _NC * _NS
_NBUF = 4
_L = 16


def _sc_body(x_hbm, p_hbm, o_hbm, pos_v, in_v, out_v, *sems):
    row = pos_v.shape[0]
    rpw = x_hbm.shape[0] // _NW
    in_sems, out_sems = sems[:_NBUF], sems[_NBUF:]
    wid = lax.axis_index("s") * _NC + lax.axis_index("c")
    base = wid * rpw

    pltpu.sync_copy(p_hbm, pos_v)

    def in_copy(r, b):
        return pltpu.make_async_copy(x_hbm.at[base + r], in_v.at[b], in_sems[b])

    def out_copy(r, b):
        return pltpu.make_async_copy(out_v.at[b], o_hbm.at[base + r], out_sems[b])

    for b in range(_NBUF):
        in_copy(b, b).start()

    n_outer = rpw // _NBUF

    def step(o, carry):
        for b in range(_NBUF):
            r = o * _NBUF + b

            @pl.when(o > 0)
            def _():
                out_copy(r - _NBUF, b).wait()

            in_copy(r, b).wait()

            @plsc.parallel_loop(0, row // _L, unroll=8)
            def _(j):
                off = j * _L
                out_v[b, pl.ds(off, _L)] = (
                    in_v[b, pl.ds(off, _L)] + pos_v[pl.ds(off, _L)])

            out_copy(r, b).start()

            @pl.when(o < n_outer - 1)
            def _():
                in_copy(r + _NBUF, b).start()
        return carry

    lax.fori_loop(0, n_outer, step, 0)

    for b in range(_NBUF):
        out_copy(rpw - _NBUF + b, b).wait()


def kernel(x, pos_table):
    B, S, D = x.shape
    row = S * D
    x2 = x.reshape(B, row)
    p1 = pos_table.reshape(row)
    mesh = plsc.VectorSubcoreMesh(core_axis_name="c", subcore_axis_name="s")
    out = pl.kernel(
        _sc_body,
        out_type=jax.ShapeDtypeStruct((B, row), jnp.float32),
        mesh=mesh,
        scratch_types=[
            pltpu.VMEM((row,), jnp.float32),
            pltpu.VMEM((_NBUF, row), jnp.float32),
            pltpu.VMEM((_NBUF, row), jnp.float32),
        ] + [pltpu.SemaphoreType.DMA] * (2 * _NBUF),
    )(x2, p1)
    return out.reshape(B, S, D)


# speedup vs baseline: 1.0013x; 1.0013x over previous
import jax
import jax.numpy as jnp
from jax import lax
from jax.experimental import pallas as pl
from jax.experimental.pallas import tpu as pltpu
from jax.experimental.pallas import tpu_sc as plsc

_NC = 2
_NS = 16
_NW = _NC * _NS
_NBUF = 4
_L = 16


def _sc_body(x_hbm, p_hbm, o_hbm, pos_v, in_v, out_v, *sems):
    row = pos_v.shape[0]
    rpw = x_hbm.shape[0] // _NW
    in_sems, out_sems = sems[:_NBUF], sems[_NBUF:]
    wid = lax.axis_index("s") * _NC + lax.axis_index("c")
    base = wid * rpw

    pltpu.sync_copy(p_hbm, pos_v)

    def in_copy(r, b):
        return pltpu.make_async_copy(x_hbm.at[base + r], in_v.at[b], in_sems[b])

    def out_copy(r, b):
        return pltpu.make_async_copy(out_v.at[b], o_hbm.at[base + r], out_sems[b])

    for b in range(_NBUF):
        in_copy(b, b).start()

    n_outer = rpw // _NBUF

    def step(o, carry):
        for b in range(_NBUF):
            r = o * _NBUF + b
            in_copy(r, b).wait()

            @pl.when((r >= 1) & (r <= rpw - _NBUF))
            def _():
                in_copy(r + _NBUF - 1, (b + _NBUF - 1) % _NBUF).start()

            @pl.when(o > 0)
            def _():
                out_copy(r - _NBUF, b).wait()

            @plsc.parallel_loop(0, row // _L, unroll=16)
            def _(j):
                off = j * _L
                out_v[b, pl.ds(off, _L)] = (
                    in_v[b, pl.ds(off, _L)] + pos_v[pl.ds(off, _L)])

            out_copy(r, b).start()
        return carry

    lax.fori_loop(0, n_outer, step, 0)

    for b in range(_NBUF):
        out_copy(rpw - _NBUF + b, b).wait()


def kernel(x, pos_table):
    B, S, D = x.shape
    row = S * D
    x2 = x.reshape(B, row)
    p1 = pos_table.reshape(row)
    mesh = plsc.VectorSubcoreMesh(core_axis_name="c", subcore_axis_name="s")
    out = pl.kernel(
        _sc_body,
        out_type=jax.ShapeDtypeStruct((B, row), jnp.float32),
        mesh=mesh,
        scratch_types=[
            pltpu.VMEM((row,), jnp.float32),
            pltpu.VMEM((_NBUF, row), jnp.float32),
            pltpu.VMEM((_NBUF, row), jnp.float32),
        ] + [pltpu.SemaphoreType.DMA] * (2 * _NBUF),
    )(x2, p1)
    return out.reshape(B, S, D)


# speedup vs baseline: 1.0525x; 1.0511x over previous
import jax
import jax.numpy as jnp
from jax import lax
from jax.experimental import pallas as pl
from jax.experimental.pallas import tpu as pltpu
from jax.experimental.pallas import tpu_sc as plsc

_NC = 2
_NS = 16
_NW = _NC * _NS
_NBUF = 4
_L = 16


def _sc_body(x_hbm, p_hbm, o_hbm, pos_v, in_v, out_v, *sems):
    row = pos_v.shape[0]
    rpw = x_hbm.shape[0] // _NW
    in_sems, out_sems = sems[:_NBUF], sems[_NBUF:]
    wid = lax.axis_index("s") * _NC + lax.axis_index("c")
    base = wid * rpw

    pltpu.sync_copy(p_hbm, pos_v)

    def in_copy(r, b):
        return pltpu.make_async_copy(x_hbm.at[base + r], in_v.at[b], in_sems[b])

    def out_copy(r, b):
        return pltpu.make_async_copy(out_v.at[b], o_hbm.at[base + r], out_sems[b])

    for b in range(_NBUF):
        in_copy(b, b).start()

    n_outer = rpw // _NBUF

    def step(o, carry):
        for b in range(0, _NBUF, 2):
            b1 = b + 1
            r = o * _NBUF + b
            r1 = r + 1
            in_copy(r, b).wait()
            in_copy(r1, b1).wait()

            @pl.when(o > 0)
            def _():
                out_copy(r - _NBUF, b).wait()
                out_copy(r1 - _NBUF, b1).wait()

            @plsc.parallel_loop(0, row // _L, unroll=8)
            def _(j):
                off = j * _L
                pv = pos_v[pl.ds(off, _L)]
                out_v[b, pl.ds(off, _L)] = in_v[b, pl.ds(off, _L)] + pv
                out_v[b1, pl.ds(off, _L)] = in_v[b1, pl.ds(off, _L)] + pv

            out_copy(r, b).start()
            out_copy(r1, b1).start()

            @pl.when(o < n_outer - 1)
            def _():
                in_copy(r + _NBUF, b).start()
                in_copy(r1 + _NBUF, b1).start()
        return carry

    lax.fori_loop(0, n_outer, step, 0)

    for b in range(_NBUF):
        out_copy(rpw - _NBUF + b, b).wait()


def kernel(x, pos_table):
    B, S, D = x.shape
    row = S * D
    x2 = x.reshape(B, row)
    p1 = pos_table.reshape(row)
    mesh = plsc.VectorSubcoreMesh(core_axis_name="c", subcore_axis_name="s")
    out = pl.kernel(
        _sc_body,
        out_type=jax.ShapeDtypeStruct((B, row), jnp.float32),
        mesh=mesh,
        scratch_types=[
            pltpu.VMEM((row,), jnp.float32),
            pltpu.VMEM((_NBUF, row), jnp.float32),
            pltpu.VMEM((_NBUF, row), jnp.float32),
        ] + [pltpu.SemaphoreType.DMA] * (2 * _NBUF),
    )(x2, p1)
    return out.reshape(B, S, D)


# speedup vs baseline: 1.0599x; 1.0071x over previous
import jax
import jax.numpy as jnp
from jax import lax
from jax.experimental import pallas as pl
from jax.experimental.pallas import tpu as pltpu
from jax.experimental.pallas import tpu_sc as plsc

_NC = 2
_NS = 16
_NW = _NC * _NS
_NBUF = 4
_L = 16


def _sc_body(x_hbm, p_hbm, o_hbm, pos_v, in_v, out_v, *sems):
    row = pos_v.shape[0]
    rpw = x_hbm.shape[0] // _NW
    in_sems, out_sems = sems[:_NBUF], sems[_NBUF:]
    wid = lax.axis_index("s") * _NC + lax.axis_index("c")
    base = wid * rpw

    pltpu.sync_copy(p_hbm, pos_v)

    def in_copy(r, b):
        return pltpu.make_async_copy(x_hbm.at[base + r], in_v.at[b], in_sems[b])

    def out_copy(r, b):
        return pltpu.make_async_copy(out_v.at[b], o_hbm.at[base + r], out_sems[b])

    for b in range(_NBUF):
        in_copy(b, b).start()

    n_outer = rpw // _NBUF

    def step(o, carry):
        for b in range(0, _NBUF, 2):
            b1 = b + 1
            r = o * _NBUF + b
            r1 = r + 1
            in_copy(r, b).wait()
            in_copy(r1, b1).wait()

            @pl.when(o > 0)
            def _():
                out_copy(r - _NBUF, b).wait()
                out_copy(r1 - _NBUF, b1).wait()

            half = row // 2

            @plsc.parallel_loop(0, half // _L, unroll=8)
            def _(j):
                off = j * _L
                pv = pos_v[pl.ds(off, _L)]
                out_v[b, pl.ds(off, _L)] = in_v[b, pl.ds(off, _L)] + pv
                out_v[b1, pl.ds(off, _L)] = in_v[b1, pl.ds(off, _L)] + pv

            pltpu.make_async_copy(
                out_v.at[b, pl.ds(0, half)],
                o_hbm.at[base + r, pl.ds(0, half)], out_sems[b]).start()
            pltpu.make_async_copy(
                out_v.at[b1, pl.ds(0, half)],
                o_hbm.at[base + r1, pl.ds(0, half)], out_sems[b1]).start()

            @plsc.parallel_loop(half // _L, row // _L, unroll=8)
            def _(j):
                off = j * _L
                pv = pos_v[pl.ds(off, _L)]
                out_v[b, pl.ds(off, _L)] = in_v[b, pl.ds(off, _L)] + pv
                out_v[b1, pl.ds(off, _L)] = in_v[b1, pl.ds(off, _L)] + pv

            pltpu.make_async_copy(
                out_v.at[b, pl.ds(half, half)],
                o_hbm.at[base + r, pl.ds(half, half)], out_sems[b]).start()
            pltpu.make_async_copy(
                out_v.at[b1, pl.ds(half, half)],
                o_hbm.at[base + r1, pl.ds(half, half)], out_sems[b1]).start()

            @pl.when(o < n_outer - 1)
            def _():
                in_copy(r + _NBUF, b).start()
                in_copy(r1 + _NBUF, b1).start()
        return carry

    lax.fori_loop(0, n_outer, step, 0)

    for b in range(_NBUF):
        out_copy(rpw - _NBUF + b, b).wait()


def kernel(x, pos_table):
    B, S, D = x.shape
    row = S * D
    x2 = x.reshape(B, row)
    p1 = pos_table.reshape(row)
    mesh = plsc.VectorSubcoreMesh(core_axis_name="c", subcore_axis_name="s")
    out = pl.kernel(
        _sc_body,
        out_type=jax.ShapeDtypeStruct((B, row), jnp.float32),
        mesh=mesh,
        scratch_types=[
            pltpu.VMEM((row,), jnp.float32),
            pltpu.VMEM((_NBUF, row), jnp.float32),
            pltpu.VMEM((_NBUF, row), jnp.float32),
        ] + [pltpu.SemaphoreType.DMA] * (2 * _NBUF),
    )(x2, p1)
    return out.reshape(B, S, D)


# speedup vs baseline: 1.1918x; 1.1244x over previous
import jax
import jax.numpy as jnp
from jax import lax
from jax.experimental import pallas as pl
from jax.experimental.pallas import tpu as pltpu
from jax.experimental.pallas import tpu_sc as plsc

_NC = 2
_NS = 16
_NW = _NC * _NS
_CH = 2
_NBUF = 2
_L = 16


def _sc_body(x_hbm, p_hbm, o_hbm, pos_v, in_v, out_v, *sems):
    row = pos_v.shape[0]
    rpw = x_hbm.shape[0] // _NW
    nch = rpw // _CH
    in_sems, out_sems = sems[:_NBUF], sems[_NBUF:]
    wid = lax.axis_index("s") * _NC + lax.axis_index("c")
    base = wid * rpw

    pltpu.sync_copy(p_hbm, pos_v)

    def in_copy(c, b):
        return pltpu.make_async_copy(
            x_hbm.at[pl.ds(base + c * _CH, _CH)], in_v.at[b], in_sems[b])

    def out_copy(c, b):
        return pltpu.make_async_copy(
            out_v.at[b], o_hbm.at[pl.ds(base + c * _CH, _CH)], out_sems[b])

    for b in range(_NBUF):
        in_copy(b, b).start()

    n_outer = nch // _NBUF

    def step(o, carry):
        for b in range(_NBUF):
            c = o * _NBUF + b
            in_copy(c, b).wait()

            @pl.when(o > 0)
            def _():
                out_copy(c - _NBUF, b).wait()

            @plsc.parallel_loop(0, row // _L, unroll=8)
            def _(j):
                off = j * _L
                pv = pos_v[pl.ds(off, _L)]
                out_v[b, 0, pl.ds(off, _L)] = in_v[b, 0, pl.ds(off, _L)] + pv
                out_v[b, 1, pl.ds(off, _L)] = in_v[b, 1, pl.ds(off, _L)] + pv

            out_copy(c, b).start()

            @pl.when(o < n_outer - 1)
            def _():
                in_copy(c + _NBUF, b).start()
        return carry

    lax.fori_loop(0, n_outer, step, 0)

    for b in range(_NBUF):
        out_copy(nch - _NBUF + b, b).wait()


def kernel(x, pos_table):
    B, S, D = x.shape
    row = S * D
    x2 = x.reshape(B, row)
    p1 = pos_table.reshape(row)
    mesh = plsc.VectorSubcoreMesh(core_axis_name="c", subcore_axis_name="s")
    out = pl.kernel(
        _sc_body,
        out_type=jax.ShapeDtypeStruct((B, row), jnp.float32),
        mesh=mesh,
        scratch_types=[
            pltpu.VMEM((row,), jnp.float32),
            pltpu.VMEM((_NBUF, _CH, row), jnp.float32),
            pltpu.VMEM((_NBUF, _CH, row), jnp.float32),
        ] + [pltpu.SemaphoreType.DMA] * (2 * _NBUF),
    )(x2, p1)
    return out.reshape(B, S, D)
